# baseline (speedup 1.0000x reference)
import functools

import jax
import jax.numpy as jnp
from jax import lax
from jax.experimental import pallas as pl
from jax.experimental.pallas import tpu as pltpu
from jax.experimental.pallas import tpu_sc as plsc

N = 10000
E = 320000
D = 128

NC = 2
NS = 16
NW = NC * NS
CH = 64
NCH = 160
GRP = 16
NG = NCH // GRP

EPT = NCH * CH
E_PAD = EPT * NW
PAD_DST = N

NCHT = E_PAD // CH
K0 = 176
K1 = NCHT // NS - K0
NG0 = K0 // GRP
NG1 = K1 // GRP
NCH0T = NS * K0

NA = 10240
RPT = NA // NS

NP = NA
BLK = 512
GRID = NP // BLK

_mesh = plsc.VectorSubcoreMesh(core_axis_name="c", subcore_axis_name="s")


@functools.partial(
    pl.kernel,
    out_type=jax.ShapeDtypeStruct((NW, NA), jnp.float32),
    mesh=_mesh,
    scratch_types=[
        pltpu.VMEM((GRP, CH), jnp.int32),
        pltpu.VMEM((NA,), jnp.float32),
    ],
    compiler_params=pltpu.CompilerParams(needs_layout_passes=False),
)
def _deg_kernel(dst_hbm, degp_hbm, dst_v, deg_v):
    c = lax.axis_index("c")
    s = lax.axis_index("s")
    wid = c * NS + s

    zeros16 = jnp.zeros((16,), jnp.float32)

    def zero_body(i, carry):
        deg_v[pl.ds(pl.multiple_of(i * 16, 16), 16)] = zeros16
        return carry

    lax.fori_loop(0, NA // 16, zero_body, 0)

    ones16 = jnp.ones((16,), jnp.float32)

    def grp_body(g, carry):
        goff = pl.multiple_of(g * GRP, GRP)
        pltpu.sync_copy(dst_hbm.at[wid, pl.ds(goff, GRP)], dst_v)

        def acc_body(j, carry2):
            for i in range(CH // 16):
                idx = dst_v[j, pl.ds(i * 16, 16)]
                plsc.addupdate_scatter(deg_v, [idx], ones16)
            return carry2

        lax.fori_loop(0, GRP, acc_body, 0)
        return carry

    lax.fori_loop(0, NG, grp_body, 0)

    pltpu.sync_copy(deg_v, degp_hbm.at[wid])


def _matmul_body(x_ref, w_ref, h_ref):
    h_ref[...] = jnp.dot(x_ref[...], w_ref[...],
                         preferred_element_type=jnp.float32,
                         precision=lax.Precision.HIGHEST)


def _scale_body(h_ref, degp_ref, hs_ref):
    deg = jnp.sum(degp_ref[...], axis=0) + 1.0
    dinv = lax.rsqrt(deg)
    hs_ref[...] = h_ref[...] * dinv[:, None]


@functools.partial(
    pl.kernel,
    out_type=jax.ShapeDtypeStruct((NC, NA, D), jnp.float32),
    mesh=_mesh,
    scratch_types=[
        pltpu.VMEM((GRP * CH,), jnp.int32),
        pltpu.VMEM((GRP * CH,), jnp.int32),
        pltpu.VMEM((GRP, CH), jnp.int32),
        pltpu.VMEM((GRP, CH), jnp.int32),
        pltpu.VMEM((CH, D), jnp.float32),
        pltpu.VMEM((CH, D), jnp.float32),
        pltpu.VMEM_SHARED((NA, D), jnp.float32),
        pltpu.SemaphoreType.DMA,
        pltpu.SemaphoreType.DMA,
        pltpu.SemaphoreType.DMA,
        pltpu.SemaphoreType.DMA,
    ],
    compiler_params=pltpu.CompilerParams(needs_layout_passes=False),
)
def _scatter_kernel(hs_hbm, src_hbm, dst_hbm, accp_hbm,
                    srcA_v, srcB_v, dstA_v, dstB_v, rows0_v, rows1_v,
                    acc_sh, sem0, sem1, semA, semB):
    c = lax.axis_index("c")
    s = lax.axis_index("s")

    ng = jnp.where(c == 0, NG0, NG1)
    base = jnp.where(c == 0, s * K0, NCH0T + s * K1)

    zeros16 = jnp.zeros((16,), jnp.float32)

    def zbody(r, carry):
        for i in range(D // 16):
            rows0_v[r, pl.ds(i * 16, 16)] = zeros16
        return carry

    lax.fori_loop(0, CH, zbody, 0)
    for k in range(RPT // CH):
        pltpu.sync_copy(rows0_v, acc_sh.at[pl.ds(s * RPT + k * CH, CH)])
    plsc.subcore_barrier()

    def src_slab(g):
        off = pl.multiple_of((base + g * GRP) * CH, GRP * CH)
        return src_hbm.at[pl.ds(off, GRP * CH)]

    def dst_slab(g):
        off = pl.multiple_of(base + g * GRP, GRP)
        return dst_hbm.at[pl.ds(off, GRP)]

    def sidx(buf, j):
        return buf.at[pl.ds(pl.multiple_of(j * CH, CH), CH)]

    pltpu.async_copy(src_slab(0), srcA_v, semA)
    pltpu.async_copy(dst_slab(0), dstA_v, semA)
    pltpu.make_async_copy(src_slab(0), srcA_v, semA).wait()
    pltpu.make_async_copy(dst_slab(0), dstA_v, semA).wait()
    pltpu.async_copy(hs_hbm.at[sidx(srcA_v, 0)], rows0_v, sem0)
    pltpu.async_copy(hs_hbm.at[sidx(srcA_v, 1)], rows1_v, sem1)

    def do_group(g, cs_v, cd_v, ns_v, nd_v, nsem):
        @pl.when(g + 1 < ng)
        def _():
            pltpu.async_copy(src_slab(g + 1), ns_v, nsem)
            pltpu.async_copy(dst_slab(g + 1), nd_v, nsem)

        def pair_body(p, carry):
            j0 = p * 2
            pltpu.make_async_copy(
                hs_hbm.at[sidx(cs_v, j0)], rows0_v, sem0).wait()
            pltpu.sync_copy(rows0_v, acc_sh.at[cd_v.at[j0]], add=True)

            @pl.when(p < GRP // 2 - 1)
            def _():
                pltpu.async_copy(hs_hbm.at[sidx(cs_v, j0 + 2)], rows0_v, sem0)

            @pl.when(jnp.logical_and(p == GRP // 2 - 1, g + 1 < ng))
            def _():
                pltpu.make_async_copy(src_slab(g + 1), ns_v, nsem).wait()
                pltpu.make_async_copy(dst_slab(g + 1), nd_v, nsem).wait()
                pltpu.async_copy(hs_hbm.at[sidx(ns_v, 0)], rows0_v, sem0)

            pltpu.make_async_copy(
                hs_hbm.at[sidx(cs_v, j0 + 1)], rows1_v, sem1).wait()
            pltpu.sync_copy(rows1_v, acc_sh.at[cd_v.at[j0 + 1]], add=True)

            @pl.when(p < GRP // 2 - 1)
            def _():
                pltpu.async_copy(hs_hbm.at[sidx(cs_v, j0 + 3)], rows1_v, sem1)

            @pl.when(jnp.logical_and(p == GRP // 2 - 1, g + 1 < ng))
            def _():
                pltpu.async_copy(hs_hbm.at[sidx(ns_v, 1)], rows1_v, sem1)

            return carry

        lax.fori_loop(0, GRP // 2, pair_body, 0)

    def grp_body(g, carry):
        @pl.when(g % 2 == 0)
        def _():
            do_group(g, srcA_v, dstA_v, srcB_v, dstB_v, semB)

        @pl.when(g % 2 == 1)
        def _():
            do_group(g, srcB_v, dstB_v, srcA_v, dstA_v, semA)

        return carry

    lax.fori_loop(0, ng, grp_body, 0)

    plsc.subcore_barrier()
    pltpu.sync_copy(acc_sh.at[pl.ds(s * RPT, RPT)],
                    accp_hbm.at[c, pl.ds(s * RPT, RPT)])


def _agg_body(acc0_ref, acc1_ref, hs_ref, degp_ref, b_ref, agg_ref, st_ref):
    i = pl.program_id(0)

    @pl.when(i == 0)
    def _():
        st_ref[...] = jnp.zeros_like(st_ref)

    deg = jnp.sum(degp_ref[...], axis=0) + 1.0
    dinv = lax.rsqrt(deg)
    a = (acc0_ref[...] + acc1_ref[...] + hs_ref[...]) * dinv[:, None]
    a = a + b_ref[...]
    agg_ref[...] = a
    rid = lax.broadcasted_iota(jnp.int32, (BLK, 1), 0) + i * BLK
    a_m = jnp.where(rid < N, a, 0.0)
    st_ref[0:1, :] += jnp.sum(a_m, axis=0, keepdims=True)
    st_ref[1:2, :] += jnp.sum(a_m * a_m, axis=0, keepdims=True)


def _bn_body(agg_ref, x_ref, st_ref, g_ref, bt_ref, y_ref):
    inv_n = jnp.float32(1.0 / N)
    mean = st_ref[0:1, :] * inv_n
    ex2 = st_ref[1:2, :] * inv_n
    var = ex2 - mean * mean
    rstd = lax.rsqrt(var + 1e-5)
    bn = g_ref[...] * (agg_ref[...] - mean) * rstd + bt_ref[...]
    y_ref[...] = jnp.maximum(jnp.maximum(bn, 0.0) + x_ref[...], 0.0)


def kernel(x, edge_index, W, b, gamma, beta):
    src = edge_index[0].astype(jnp.int32)
    dst = edge_index[1].astype(jnp.int32)
    pad = E_PAD - E
    src1 = jnp.concatenate([src, jnp.zeros((pad,), jnp.int32)])
    dst1 = jnp.concatenate([dst, jnp.full((pad,), PAD_DST, jnp.int32)])
    dst2 = dst1.reshape(NCHT, CH)
    dst3 = dst1.reshape(NW, NCH, CH)
    x_p = jnp.concatenate([x, jnp.zeros((NP - N, D), jnp.float32)])

    degp = _deg_kernel(dst3)

    h = pl.pallas_call(
        _matmul_body,
        grid=(GRID,),
        in_specs=[
            pl.BlockSpec((BLK, D), lambda i: (i, 0)),
            pl.BlockSpec((D, D), lambda i: (0, 0)),
        ],
        out_specs=pl.BlockSpec((BLK, D), lambda i: (i, 0)),
        out_shape=jax.ShapeDtypeStruct((NP, D), jnp.float32),
    )(x_p, W)

    hs = pl.pallas_call(
        _scale_body,
        grid=(GRID,),
        in_specs=[
            pl.BlockSpec((BLK, D), lambda i: (i, 0)),
            pl.BlockSpec((NW, BLK), lambda i: (0, i)),
        ],
        out_specs=pl.BlockSpec((BLK, D), lambda i: (i, 0)),
        out_shape=jax.ShapeDtypeStruct((NP, D), jnp.float32),
    )(h, degp)

    accp = _scatter_kernel(hs, src1, dst2)

    agg, stats = pl.pallas_call(
        _agg_body,
        grid=(GRID,),
        in_specs=[
            pl.BlockSpec((BLK, D), lambda i: (i, 0)),
            pl.BlockSpec((BLK, D), lambda i: (i, 0)),
            pl.BlockSpec((BLK, D), lambda i: (i, 0)),
            pl.BlockSpec((NW, BLK), lambda i: (0, i)),
            pl.BlockSpec((1, D), lambda i: (0, 0)),
        ],
        out_specs=[
            pl.BlockSpec((BLK, D), lambda i: (i, 0)),
            pl.BlockSpec((2, D), lambda i: (0, 0)),
        ],
        out_shape=[
            jax.ShapeDtypeStruct((NP, D), jnp.float32),
            jax.ShapeDtypeStruct((2, D), jnp.float32),
        ],
    )(accp[0], accp[1], hs, degp, b.reshape(1, D))

    y = pl.pallas_call(
        _bn_body,
        grid=(GRID,),
        in_specs=[
            pl.BlockSpec((BLK, D), lambda i: (i, 0)),
            pl.BlockSpec((BLK, D), lambda i: (i, 0)),
            pl.BlockSpec((2, D), lambda i: (0, 0)),
            pl.BlockSpec((1, D), lambda i: (0, 0)),
            pl.BlockSpec((1, D), lambda i: (0, 0)),
        ],
        out_specs=pl.BlockSpec((BLK, D), lambda i: (i, 0)),
        out_shape=jax.ShapeDtypeStruct((NP, D), jnp.float32),
    )(agg, x_p, stats, gamma.reshape(1, D), beta.reshape(1, D))

    return y[:N]


# speedup vs baseline: 1.0314x; 1.0314x over previous
import functools

import jax
import jax.numpy as jnp
from jax import lax
from jax.experimental import pallas as pl
from jax.experimental.pallas import tpu as pltpu
from jax.experimental.pallas import tpu_sc as plsc

N = 10000
E = 320000
D = 128

NC = 2
NS = 16
NW = NC * NS
CH = 64
NCH = 160
GRP = 16
NG = NCH // GRP

EPT = NCH * CH
E_PAD = EPT * NW
PAD_DST = N

NCHT = E_PAD // CH
K0 = 208
K1 = NCHT // NS - K0
NG0 = K0 // GRP
NG1 = K1 // GRP
NCH0T = NS * K0

NA = 10240
RPT = NA // NS

NP = NA
BLK = 512
GRID = NP // BLK

_mesh = plsc.VectorSubcoreMesh(core_axis_name="c", subcore_axis_name="s")


@functools.partial(
    pl.kernel,
    out_type=jax.ShapeDtypeStruct((NW, NA), jnp.float32),
    mesh=_mesh,
    scratch_types=[
        pltpu.VMEM((GRP, CH), jnp.int32),
        pltpu.VMEM((NA,), jnp.float32),
    ],
    compiler_params=pltpu.CompilerParams(needs_layout_passes=False),
)
def _deg_kernel(dst_hbm, degp_hbm, dst_v, deg_v):
    c = lax.axis_index("c")
    s = lax.axis_index("s")
    wid = c * NS + s

    zeros16 = jnp.zeros((16,), jnp.float32)

    def zero_body(i, carry):
        deg_v[pl.ds(pl.multiple_of(i * 16, 16), 16)] = zeros16
        return carry

    lax.fori_loop(0, NA // 16, zero_body, 0)

    ones16 = jnp.ones((16,), jnp.float32)

    def grp_body(g, carry):
        goff = pl.multiple_of(g * GRP, GRP)
        pltpu.sync_copy(dst_hbm.at[wid, pl.ds(goff, GRP)], dst_v)

        def acc_body(j, carry2):
            for i in range(CH // 16):
                idx = dst_v[j, pl.ds(i * 16, 16)]
                plsc.addupdate_scatter(deg_v, [idx], ones16)
            return carry2

        lax.fori_loop(0, GRP, acc_body, 0)
        return carry

    lax.fori_loop(0, NG, grp_body, 0)

    pltpu.sync_copy(deg_v, degp_hbm.at[wid])


def _matmul_body(x_ref, w_ref, h_ref):
    h_ref[...] = jnp.dot(x_ref[...], w_ref[...],
                         preferred_element_type=jnp.float32,
                         precision=lax.Precision.HIGHEST)


def _scale_body(h_ref, degp_ref, hs_ref):
    deg = jnp.sum(degp_ref[...], axis=0) + 1.0
    dinv = lax.rsqrt(deg)
    hs_ref[...] = h_ref[...] * dinv[:, None]


@functools.partial(
    pl.kernel,
    out_type=jax.ShapeDtypeStruct((NC, NA, D), jnp.float32),
    mesh=_mesh,
    scratch_types=[
        pltpu.VMEM((GRP * CH,), jnp.int32),
        pltpu.VMEM((GRP * CH,), jnp.int32),
        pltpu.VMEM((GRP, CH), jnp.int32),
        pltpu.VMEM((GRP, CH), jnp.int32),
        pltpu.VMEM((CH, D), jnp.float32),
        pltpu.VMEM((CH, D), jnp.float32),
        pltpu.VMEM_SHARED((NA, D), jnp.float32),
        pltpu.SemaphoreType.DMA,
        pltpu.SemaphoreType.DMA,
        pltpu.SemaphoreType.DMA,
        pltpu.SemaphoreType.DMA,
    ],
    compiler_params=pltpu.CompilerParams(needs_layout_passes=False),
)
def _scatter_kernel(hs_hbm, src_hbm, dst_hbm, accp_hbm,
                    srcA_v, srcB_v, dstA_v, dstB_v, rows0_v, rows1_v,
                    acc_sh, sem0, sem1, semA, semB):
    c = lax.axis_index("c")
    s = lax.axis_index("s")

    ng = jnp.where(c == 0, NG0, NG1)
    base = jnp.where(c == 0, s * K0, NCH0T + s * K1)

    zeros16 = jnp.zeros((16,), jnp.float32)

    def zbody(r, carry):
        for i in range(D // 16):
            rows0_v[r, pl.ds(i * 16, 16)] = zeros16
        return carry

    lax.fori_loop(0, CH, zbody, 0)
    for k in range(RPT // CH):
        pltpu.sync_copy(rows0_v, acc_sh.at[pl.ds(s * RPT + k * CH, CH)])
    plsc.subcore_barrier()

    def src_slab(g):
        off = pl.multiple_of((base + g * GRP) * CH, GRP * CH)
        return src_hbm.at[pl.ds(off, GRP * CH)]

    def dst_slab(g):
        off = pl.multiple_of(base + g * GRP, GRP)
        return dst_hbm.at[pl.ds(off, GRP)]

    def sidx(buf, j):
        return buf.at[pl.ds(pl.multiple_of(j * CH, CH), CH)]

    pltpu.async_copy(src_slab(0), srcA_v, semA)
    pltpu.async_copy(dst_slab(0), dstA_v, semA)
    pltpu.make_async_copy(src_slab(0), srcA_v, semA).wait()
    pltpu.make_async_copy(dst_slab(0), dstA_v, semA).wait()
    pltpu.async_copy(hs_hbm.at[sidx(srcA_v, 0)], rows0_v, sem0)
    pltpu.async_copy(hs_hbm.at[sidx(srcA_v, 1)], rows1_v, sem1)

    def do_group(g, cs_v, cd_v, ns_v, nd_v, nsem):
        @pl.when(g + 1 < ng)
        def _():
            pltpu.async_copy(src_slab(g + 1), ns_v, nsem)
            pltpu.async_copy(dst_slab(g + 1), nd_v, nsem)

        def pair_body(p, carry):
            j0 = p * 2
            pltpu.make_async_copy(
                hs_hbm.at[sidx(cs_v, j0)], rows0_v, sem0).wait()
            pltpu.sync_copy(rows0_v, acc_sh.at[cd_v.at[j0]], add=True)

            @pl.when(p < GRP // 2 - 1)
            def _():
                pltpu.async_copy(hs_hbm.at[sidx(cs_v, j0 + 2)], rows0_v, sem0)

            @pl.when(jnp.logical_and(p == GRP // 2 - 1, g + 1 < ng))
            def _():
                pltpu.make_async_copy(src_slab(g + 1), ns_v, nsem).wait()
                pltpu.make_async_copy(dst_slab(g + 1), nd_v, nsem).wait()
                pltpu.async_copy(hs_hbm.at[sidx(ns_v, 0)], rows0_v, sem0)

            pltpu.make_async_copy(
                hs_hbm.at[sidx(cs_v, j0 + 1)], rows1_v, sem1).wait()
            pltpu.sync_copy(rows1_v, acc_sh.at[cd_v.at[j0 + 1]], add=True)

            @pl.when(p < GRP // 2 - 1)
            def _():
                pltpu.async_copy(hs_hbm.at[sidx(cs_v, j0 + 3)], rows1_v, sem1)

            @pl.when(jnp.logical_and(p == GRP // 2 - 1, g + 1 < ng))
            def _():
                pltpu.async_copy(hs_hbm.at[sidx(ns_v, 1)], rows1_v, sem1)

            return carry

        lax.fori_loop(0, GRP // 2, pair_body, 0)

    def grp_body(g, carry):
        @pl.when(g % 2 == 0)
        def _():
            do_group(g, srcA_v, dstA_v, srcB_v, dstB_v, semB)

        @pl.when(g % 2 == 1)
        def _():
            do_group(g, srcB_v, dstB_v, srcA_v, dstA_v, semA)

        return carry

    lax.fori_loop(0, ng, grp_body, 0)

    plsc.subcore_barrier()
    pltpu.sync_copy(acc_sh.at[pl.ds(s * RPT, RPT)],
                    accp_hbm.at[c, pl.ds(s * RPT, RPT)])


def _agg_body(acc0_ref, acc1_ref, hs_ref, degp_ref, b_ref, agg_ref, st_ref):
    i = pl.program_id(0)

    @pl.when(i == 0)
    def _():
        st_ref[...] = jnp.zeros_like(st_ref)

    deg = jnp.sum(degp_ref[...], axis=0) + 1.0
    dinv = lax.rsqrt(deg)
    a = (acc0_ref[...] + acc1_ref[...] + hs_ref[...]) * dinv[:, None]
    a = a + b_ref[...]
    agg_ref[...] = a
    rid = lax.broadcasted_iota(jnp.int32, (BLK, 1), 0) + i * BLK
    a_m = jnp.where(rid < N, a, 0.0)
    st_ref[0:1, :] += jnp.sum(a_m, axis=0, keepdims=True)
    st_ref[1:2, :] += jnp.sum(a_m * a_m, axis=0, keepdims=True)


def _bn_body(agg_ref, x_ref, st_ref, g_ref, bt_ref, y_ref):
    inv_n = jnp.float32(1.0 / N)
    mean = st_ref[0:1, :] * inv_n
    ex2 = st_ref[1:2, :] * inv_n
    var = ex2 - mean * mean
    rstd = lax.rsqrt(var + 1e-5)
    bn = g_ref[...] * (agg_ref[...] - mean) * rstd + bt_ref[...]
    y_ref[...] = jnp.maximum(jnp.maximum(bn, 0.0) + x_ref[...], 0.0)


def kernel(x, edge_index, W, b, gamma, beta):
    src = edge_index[0].astype(jnp.int32)
    dst = edge_index[1].astype(jnp.int32)
    pad = E_PAD - E
    src1 = jnp.concatenate([src, jnp.zeros((pad,), jnp.int32)])
    dst1 = jnp.concatenate([dst, jnp.full((pad,), PAD_DST, jnp.int32)])
    dst2 = dst1.reshape(NCHT, CH)
    dst3 = dst1.reshape(NW, NCH, CH)
    x_p = jnp.concatenate([x, jnp.zeros((NP - N, D), jnp.float32)])

    degp = _deg_kernel(dst3)

    h = pl.pallas_call(
        _matmul_body,
        grid=(GRID,),
        in_specs=[
            pl.BlockSpec((BLK, D), lambda i: (i, 0)),
            pl.BlockSpec((D, D), lambda i: (0, 0)),
        ],
        out_specs=pl.BlockSpec((BLK, D), lambda i: (i, 0)),
        out_shape=jax.ShapeDtypeStruct((NP, D), jnp.float32),
    )(x_p, W)

    hs = pl.pallas_call(
        _scale_body,
        grid=(GRID,),
        in_specs=[
            pl.BlockSpec((BLK, D), lambda i: (i, 0)),
            pl.BlockSpec((NW, BLK), lambda i: (0, i)),
        ],
        out_specs=pl.BlockSpec((BLK, D), lambda i: (i, 0)),
        out_shape=jax.ShapeDtypeStruct((NP, D), jnp.float32),
    )(h, degp)

    accp = _scatter_kernel(hs, src1, dst2)

    agg, stats = pl.pallas_call(
        _agg_body,
        grid=(GRID,),
        in_specs=[
            pl.BlockSpec((BLK, D), lambda i: (i, 0)),
            pl.BlockSpec((BLK, D), lambda i: (i, 0)),
            pl.BlockSpec((BLK, D), lambda i: (i, 0)),
            pl.BlockSpec((NW, BLK), lambda i: (0, i)),
            pl.BlockSpec((1, D), lambda i: (0, 0)),
        ],
        out_specs=[
            pl.BlockSpec((BLK, D), lambda i: (i, 0)),
            pl.BlockSpec((2, D), lambda i: (0, 0)),
        ],
        out_shape=[
            jax.ShapeDtypeStruct((NP, D), jnp.float32),
            jax.ShapeDtypeStruct((2, D), jnp.float32),
        ],
    )(accp[0], accp[1], hs, degp, b.reshape(1, D))

    y = pl.pallas_call(
        _bn_body,
        grid=(GRID,),
        in_specs=[
            pl.BlockSpec((BLK, D), lambda i: (i, 0)),
            pl.BlockSpec((BLK, D), lambda i: (i, 0)),
            pl.BlockSpec((2, D), lambda i: (0, 0)),
            pl.BlockSpec((1, D), lambda i: (0, 0)),
            pl.BlockSpec((1, D), lambda i: (0, 0)),
        ],
        out_specs=pl.BlockSpec((BLK, D), lambda i: (i, 0)),
        out_shape=jax.ShapeDtypeStruct((NP, D), jnp.float32),
    )(agg, x_p, stats, gamma.reshape(1, D), beta.reshape(1, D))

    return y[:N]


# speedup vs baseline: 1.0441x; 1.0123x over previous
import functools

import jax
import jax.numpy as jnp
from jax import lax
from jax.experimental import pallas as pl
from jax.experimental.pallas import tpu as pltpu
from jax.experimental.pallas import tpu_sc as plsc

N = 10000
E = 320000
D = 128

NC = 2
NS = 16
NW = NC * NS
CH = 64
NCH = 160
GRP = 16
NG = NCH // GRP

EPT = NCH * CH
E_PAD = EPT * NW
PAD_DST = N

NCHT = E_PAD // CH
K0 = 240
K1 = NCHT // NS - K0
NG0 = K0 // GRP
NG1 = K1 // GRP
NCH0T = NS * K0

NA = 10240
RPT = NA // NS

NP = NA
BLK = 512
GRID = NP // BLK

_mesh = plsc.VectorSubcoreMesh(core_axis_name="c", subcore_axis_name="s")


@functools.partial(
    pl.kernel,
    out_type=jax.ShapeDtypeStruct((NW, NA), jnp.float32),
    mesh=_mesh,
    scratch_types=[
        pltpu.VMEM((GRP, CH), jnp.int32),
        pltpu.VMEM((NA,), jnp.float32),
    ],
    compiler_params=pltpu.CompilerParams(needs_layout_passes=False),
)
def _deg_kernel(dst_hbm, degp_hbm, dst_v, deg_v):
    c = lax.axis_index("c")
    s = lax.axis_index("s")
    wid = c * NS + s

    zeros16 = jnp.zeros((16,), jnp.float32)

    def zero_body(i, carry):
        deg_v[pl.ds(pl.multiple_of(i * 16, 16), 16)] = zeros16
        return carry

    lax.fori_loop(0, NA // 16, zero_body, 0)

    ones16 = jnp.ones((16,), jnp.float32)

    def grp_body(g, carry):
        goff = pl.multiple_of(g * GRP, GRP)
        pltpu.sync_copy(dst_hbm.at[wid, pl.ds(goff, GRP)], dst_v)

        def acc_body(j, carry2):
            for i in range(CH // 16):
                idx = dst_v[j, pl.ds(i * 16, 16)]
                plsc.addupdate_scatter(deg_v, [idx], ones16)
            return carry2

        lax.fori_loop(0, GRP, acc_body, 0)
        return carry

    lax.fori_loop(0, NG, grp_body, 0)

    pltpu.sync_copy(deg_v, degp_hbm.at[wid])


def _matmul_body(x_ref, w_ref, h_ref):
    h_ref[...] = jnp.dot(x_ref[...], w_ref[...],
                         preferred_element_type=jnp.float32,
                         precision=lax.Precision.HIGHEST)


def _scale_body(h_ref, degp_ref, hs_ref):
    deg = jnp.sum(degp_ref[...], axis=0) + 1.0
    dinv = lax.rsqrt(deg)
    hs_ref[...] = h_ref[...] * dinv[:, None]


@functools.partial(
    pl.kernel,
    out_type=jax.ShapeDtypeStruct((NC, NA, D), jnp.float32),
    mesh=_mesh,
    scratch_types=[
        pltpu.VMEM((GRP * CH,), jnp.int32),
        pltpu.VMEM((GRP * CH,), jnp.int32),
        pltpu.VMEM((GRP, CH), jnp.int32),
        pltpu.VMEM((GRP, CH), jnp.int32),
        pltpu.VMEM((CH, D), jnp.float32),
        pltpu.VMEM((CH, D), jnp.float32),
        pltpu.VMEM_SHARED((NA, D), jnp.float32),
        pltpu.SemaphoreType.DMA,
        pltpu.SemaphoreType.DMA,
        pltpu.SemaphoreType.DMA,
        pltpu.SemaphoreType.DMA,
    ],
    compiler_params=pltpu.CompilerParams(needs_layout_passes=False),
)
def _scatter_kernel(hs_hbm, src_hbm, dst_hbm, accp_hbm,
                    srcA_v, srcB_v, dstA_v, dstB_v, rows0_v, rows1_v,
                    acc_sh, sem0, sem1, semA, semB):
    c = lax.axis_index("c")
    s = lax.axis_index("s")

    ng = jnp.where(c == 0, NG0, NG1)
    base = jnp.where(c == 0, s * K0, NCH0T + s * K1)

    zeros16 = jnp.zeros((16,), jnp.float32)

    def zbody(r, carry):
        for i in range(D // 16):
            rows0_v[r, pl.ds(i * 16, 16)] = zeros16
        return carry

    lax.fori_loop(0, CH, zbody, 0)
    for k in range(RPT // CH):
        pltpu.sync_copy(rows0_v, acc_sh.at[pl.ds(s * RPT + k * CH, CH)])
    plsc.subcore_barrier()

    def src_slab(g):
        off = pl.multiple_of((base + g * GRP) * CH, GRP * CH)
        return src_hbm.at[pl.ds(off, GRP * CH)]

    def dst_slab(g):
        off = pl.multiple_of(base + g * GRP, GRP)
        return dst_hbm.at[pl.ds(off, GRP)]

    def sidx(buf, j):
        return buf.at[pl.ds(pl.multiple_of(j * CH, CH), CH)]

    pltpu.async_copy(src_slab(0), srcA_v, semA)
    pltpu.async_copy(dst_slab(0), dstA_v, semA)
    pltpu.make_async_copy(src_slab(0), srcA_v, semA).wait()
    pltpu.make_async_copy(dst_slab(0), dstA_v, semA).wait()
    pltpu.async_copy(hs_hbm.at[sidx(srcA_v, 0)], rows0_v, sem0)
    pltpu.async_copy(hs_hbm.at[sidx(srcA_v, 1)], rows1_v, sem1)

    def do_group(g, cs_v, cd_v, ns_v, nd_v, nsem):
        @pl.when(g + 1 < ng)
        def _():
            pltpu.async_copy(src_slab(g + 1), ns_v, nsem)
            pltpu.async_copy(dst_slab(g + 1), nd_v, nsem)

        def pair_body(p, carry):
            j0 = p * 2
            pltpu.make_async_copy(
                hs_hbm.at[sidx(cs_v, j0)], rows0_v, sem0).wait()
            pltpu.sync_copy(rows0_v, acc_sh.at[cd_v.at[j0]], add=True)

            @pl.when(p < GRP // 2 - 1)
            def _():
                pltpu.async_copy(hs_hbm.at[sidx(cs_v, j0 + 2)], rows0_v, sem0)

            @pl.when(jnp.logical_and(p == GRP // 2 - 1, g + 1 < ng))
            def _():
                pltpu.make_async_copy(src_slab(g + 1), ns_v, nsem).wait()
                pltpu.make_async_copy(dst_slab(g + 1), nd_v, nsem).wait()
                pltpu.async_copy(hs_hbm.at[sidx(ns_v, 0)], rows0_v, sem0)

            pltpu.make_async_copy(
                hs_hbm.at[sidx(cs_v, j0 + 1)], rows1_v, sem1).wait()
            pltpu.sync_copy(rows1_v, acc_sh.at[cd_v.at[j0 + 1]], add=True)

            @pl.when(p < GRP // 2 - 1)
            def _():
                pltpu.async_copy(hs_hbm.at[sidx(cs_v, j0 + 3)], rows1_v, sem1)

            @pl.when(jnp.logical_and(p == GRP // 2 - 1, g + 1 < ng))
            def _():
                pltpu.async_copy(hs_hbm.at[sidx(ns_v, 1)], rows1_v, sem1)

            return carry

        lax.fori_loop(0, GRP // 2, pair_body, 0)

    def grp_body(g, carry):
        @pl.when(g % 2 == 0)
        def _():
            do_group(g, srcA_v, dstA_v, srcB_v, dstB_v, semB)

        @pl.when(g % 2 == 1)
        def _():
            do_group(g, srcB_v, dstB_v, srcA_v, dstA_v, semA)

        return carry

    lax.fori_loop(0, ng, grp_body, 0)

    plsc.subcore_barrier()
    pltpu.sync_copy(acc_sh.at[pl.ds(s * RPT, RPT)],
                    accp_hbm.at[c, pl.ds(s * RPT, RPT)])


def _agg_body(acc0_ref, acc1_ref, hs_ref, degp_ref, b_ref, agg_ref, st_ref):
    i = pl.program_id(0)

    @pl.when(i == 0)
    def _():
        st_ref[...] = jnp.zeros_like(st_ref)

    deg = jnp.sum(degp_ref[...], axis=0) + 1.0
    dinv = lax.rsqrt(deg)
    a = (acc0_ref[...] + acc1_ref[...] + hs_ref[...]) * dinv[:, None]
    a = a + b_ref[...]
    agg_ref[...] = a
    rid = lax.broadcasted_iota(jnp.int32, (BLK, 1), 0) + i * BLK
    a_m = jnp.where(rid < N, a, 0.0)
    st_ref[0:1, :] += jnp.sum(a_m, axis=0, keepdims=True)
    st_ref[1:2, :] += jnp.sum(a_m * a_m, axis=0, keepdims=True)


def _bn_body(agg_ref, x_ref, st_ref, g_ref, bt_ref, y_ref):
    inv_n = jnp.float32(1.0 / N)
    mean = st_ref[0:1, :] * inv_n
    ex2 = st_ref[1:2, :] * inv_n
    var = ex2 - mean * mean
    rstd = lax.rsqrt(var + 1e-5)
    bn = g_ref[...] * (agg_ref[...] - mean) * rstd + bt_ref[...]
    y_ref[...] = jnp.maximum(jnp.maximum(bn, 0.0) + x_ref[...], 0.0)


def kernel(x, edge_index, W, b, gamma, beta):
    src = edge_index[0].astype(jnp.int32)
    dst = edge_index[1].astype(jnp.int32)
    pad = E_PAD - E
    src1 = jnp.concatenate([src, jnp.zeros((pad,), jnp.int32)])
    dst1 = jnp.concatenate([dst, jnp.full((pad,), PAD_DST, jnp.int32)])
    dst2 = dst1.reshape(NCHT, CH)
    dst3 = dst1.reshape(NW, NCH, CH)
    x_p = jnp.concatenate([x, jnp.zeros((NP - N, D), jnp.float32)])

    degp = _deg_kernel(dst3)

    h = pl.pallas_call(
        _matmul_body,
        grid=(GRID,),
        in_specs=[
            pl.BlockSpec((BLK, D), lambda i: (i, 0)),
            pl.BlockSpec((D, D), lambda i: (0, 0)),
        ],
        out_specs=pl.BlockSpec((BLK, D), lambda i: (i, 0)),
        out_shape=jax.ShapeDtypeStruct((NP, D), jnp.float32),
    )(x_p, W)

    hs = pl.pallas_call(
        _scale_body,
        grid=(GRID,),
        in_specs=[
            pl.BlockSpec((BLK, D), lambda i: (i, 0)),
            pl.BlockSpec((NW, BLK), lambda i: (0, i)),
        ],
        out_specs=pl.BlockSpec((BLK, D), lambda i: (i, 0)),
        out_shape=jax.ShapeDtypeStruct((NP, D), jnp.float32),
    )(h, degp)

    accp = _scatter_kernel(hs, src1, dst2)

    agg, stats = pl.pallas_call(
        _agg_body,
        grid=(GRID,),
        in_specs=[
            pl.BlockSpec((BLK, D), lambda i: (i, 0)),
            pl.BlockSpec((BLK, D), lambda i: (i, 0)),
            pl.BlockSpec((BLK, D), lambda i: (i, 0)),
            pl.BlockSpec((NW, BLK), lambda i: (0, i)),
            pl.BlockSpec((1, D), lambda i: (0, 0)),
        ],
        out_specs=[
            pl.BlockSpec((BLK, D), lambda i: (i, 0)),
            pl.BlockSpec((2, D), lambda i: (0, 0)),
        ],
        out_shape=[
            jax.ShapeDtypeStruct((NP, D), jnp.float32),
            jax.ShapeDtypeStruct((2, D), jnp.float32),
        ],
    )(accp[0], accp[1], hs, degp, b.reshape(1, D))

    y = pl.pallas_call(
        _bn_body,
        grid=(GRID,),
        in_specs=[
            pl.BlockSpec((BLK, D), lambda i: (i, 0)),
            pl.BlockSpec((BLK, D), lambda i: (i, 0)),
            pl.BlockSpec((2, D), lambda i: (0, 0)),
            pl.BlockSpec((1, D), lambda i: (0, 0)),
            pl.BlockSpec((1, D), lambda i: (0, 0)),
        ],
        out_specs=pl.BlockSpec((BLK, D), lambda i: (i, 0)),
        out_shape=jax.ShapeDtypeStruct((NP, D), jnp.float32),
    )(agg, x_p, stats, gamma.reshape(1, D), beta.reshape(1, D))

    return y[:N]


# speedup vs baseline: 1.0517x; 1.0073x over previous
import functools

import jax
import jax.numpy as jnp
from jax import lax
from jax.experimental import pallas as pl
from jax.experimental.pallas import tpu as pltpu
from jax.experimental.pallas import tpu_sc as plsc

N = 10000
E = 320000
D = 128

NC = 2
NS = 16
NW = NC * NS
CH = 64
NCH = 160
GRP = 16
NG = NCH // GRP

EPT = NCH * CH
E_PAD = EPT * NW
PAD_DST = N

NCHT = E_PAD // CH
K0 = 256
K1 = NCHT // NS - K0
NG0 = K0 // GRP
NG1 = K1 // GRP
NCH0T = NS * K0

NA = 10240
RPT = NA // NS

NP = NA
BLK = 512
GRID = NP // BLK

_mesh = plsc.VectorSubcoreMesh(core_axis_name="c", subcore_axis_name="s")


@functools.partial(
    pl.kernel,
    out_type=jax.ShapeDtypeStruct((NW, NA), jnp.float32),
    mesh=_mesh,
    scratch_types=[
        pltpu.VMEM((GRP, CH), jnp.int32),
        pltpu.VMEM((NA,), jnp.float32),
    ],
    compiler_params=pltpu.CompilerParams(needs_layout_passes=False),
)
def _deg_kernel(dst_hbm, degp_hbm, dst_v, deg_v):
    c = lax.axis_index("c")
    s = lax.axis_index("s")
    wid = c * NS + s

    zeros16 = jnp.zeros((16,), jnp.float32)

    def zero_body(i, carry):
        deg_v[pl.ds(pl.multiple_of(i * 16, 16), 16)] = zeros16
        return carry

    lax.fori_loop(0, NA // 16, zero_body, 0)

    ones16 = jnp.ones((16,), jnp.float32)

    def grp_body(g, carry):
        goff = pl.multiple_of(g * GRP, GRP)
        pltpu.sync_copy(dst_hbm.at[wid, pl.ds(goff, GRP)], dst_v)

        def acc_body(j, carry2):
            for i in range(CH // 16):
                idx = dst_v[j, pl.ds(i * 16, 16)]
                plsc.addupdate_scatter(deg_v, [idx], ones16)
            return carry2

        lax.fori_loop(0, GRP, acc_body, 0)
        return carry

    lax.fori_loop(0, NG, grp_body, 0)

    pltpu.sync_copy(deg_v, degp_hbm.at[wid])


def _matmul_body(x_ref, w_ref, h_ref):
    h_ref[...] = jnp.dot(x_ref[...], w_ref[...],
                         preferred_element_type=jnp.float32,
                         precision=lax.Precision.HIGHEST)


def _scale_body(h_ref, degp_ref, hs_ref):
    deg = jnp.sum(degp_ref[...], axis=0) + 1.0
    dinv = lax.rsqrt(deg)
    hs_ref[...] = h_ref[...] * dinv[:, None]


@functools.partial(
    pl.kernel,
    out_type=jax.ShapeDtypeStruct((NC, NA, D), jnp.float32),
    mesh=_mesh,
    scratch_types=[
        pltpu.VMEM((GRP * CH,), jnp.int32),
        pltpu.VMEM((GRP * CH,), jnp.int32),
        pltpu.VMEM((GRP, CH), jnp.int32),
        pltpu.VMEM((GRP, CH), jnp.int32),
        pltpu.VMEM((CH, D), jnp.float32),
        pltpu.VMEM((CH, D), jnp.float32),
        pltpu.VMEM_SHARED((NA, D), jnp.float32),
        pltpu.SemaphoreType.DMA,
        pltpu.SemaphoreType.DMA,
        pltpu.SemaphoreType.DMA,
        pltpu.SemaphoreType.DMA,
    ],
    compiler_params=pltpu.CompilerParams(needs_layout_passes=False),
)
def _scatter_kernel(hs_hbm, src_hbm, dst_hbm, accp_hbm,
                    srcA_v, srcB_v, dstA_v, dstB_v, rows0_v, rows1_v,
                    acc_sh, sem0, sem1, semA, semB):
    c = lax.axis_index("c")
    s = lax.axis_index("s")

    ng = jnp.where(c == 0, NG0, NG1)
    base = jnp.where(c == 0, s * K0, NCH0T + s * K1)

    zeros16 = jnp.zeros((16,), jnp.float32)

    def zbody(r, carry):
        for i in range(D // 16):
            rows0_v[r, pl.ds(i * 16, 16)] = zeros16
        return carry

    lax.fori_loop(0, CH, zbody, 0)
    for k in range(RPT // CH):
        pltpu.sync_copy(rows0_v, acc_sh.at[pl.ds(s * RPT + k * CH, CH)])
    plsc.subcore_barrier()

    def src_slab(g):
        off = pl.multiple_of((base + g * GRP) * CH, GRP * CH)
        return src_hbm.at[pl.ds(off, GRP * CH)]

    def dst_slab(g):
        off = pl.multiple_of(base + g * GRP, GRP)
        return dst_hbm.at[pl.ds(off, GRP)]

    def sidx(buf, j):
        return buf.at[pl.ds(pl.multiple_of(j * CH, CH), CH)]

    pltpu.async_copy(src_slab(0), srcA_v, semA)
    pltpu.async_copy(dst_slab(0), dstA_v, semA)
    pltpu.make_async_copy(src_slab(0), srcA_v, semA).wait()
    pltpu.make_async_copy(dst_slab(0), dstA_v, semA).wait()
    pltpu.async_copy(hs_hbm.at[sidx(srcA_v, 0)], rows0_v, sem0)
    pltpu.async_copy(hs_hbm.at[sidx(srcA_v, 1)], rows1_v, sem1)

    def do_group(g, cs_v, cd_v, ns_v, nd_v, nsem):
        @pl.when(g + 1 < ng)
        def _():
            pltpu.async_copy(src_slab(g + 1), ns_v, nsem)
            pltpu.async_copy(dst_slab(g + 1), nd_v, nsem)

        def pair_body(p, carry):
            j0 = p * 2
            pltpu.make_async_copy(
                hs_hbm.at[sidx(cs_v, j0)], rows0_v, sem0).wait()
            pltpu.sync_copy(rows0_v, acc_sh.at[cd_v.at[j0]], add=True)

            @pl.when(p < GRP // 2 - 1)
            def _():
                pltpu.async_copy(hs_hbm.at[sidx(cs_v, j0 + 2)], rows0_v, sem0)

            @pl.when(jnp.logical_and(p == GRP // 2 - 1, g + 1 < ng))
            def _():
                pltpu.make_async_copy(src_slab(g + 1), ns_v, nsem).wait()
                pltpu.make_async_copy(dst_slab(g + 1), nd_v, nsem).wait()
                pltpu.async_copy(hs_hbm.at[sidx(ns_v, 0)], rows0_v, sem0)

            pltpu.make_async_copy(
                hs_hbm.at[sidx(cs_v, j0 + 1)], rows1_v, sem1).wait()
            pltpu.sync_copy(rows1_v, acc_sh.at[cd_v.at[j0 + 1]], add=True)

            @pl.when(p < GRP // 2 - 1)
            def _():
                pltpu.async_copy(hs_hbm.at[sidx(cs_v, j0 + 3)], rows1_v, sem1)

            @pl.when(jnp.logical_and(p == GRP // 2 - 1, g + 1 < ng))
            def _():
                pltpu.async_copy(hs_hbm.at[sidx(ns_v, 1)], rows1_v, sem1)

            return carry

        lax.fori_loop(0, GRP // 2, pair_body, 0)

    def grp_body(g, carry):
        @pl.when(g % 2 == 0)
        def _():
            do_group(g, srcA_v, dstA_v, srcB_v, dstB_v, semB)

        @pl.when(g % 2 == 1)
        def _():
            do_group(g, srcB_v, dstB_v, srcA_v, dstA_v, semA)

        return carry

    lax.fori_loop(0, ng, grp_body, 0)

    plsc.subcore_barrier()
    pltpu.sync_copy(acc_sh.at[pl.ds(s * RPT, RPT)],
                    accp_hbm.at[c, pl.ds(s * RPT, RPT)])


def _agg_body(acc0_ref, acc1_ref, hs_ref, degp_ref, b_ref, agg_ref, st_ref):
    i = pl.program_id(0)

    @pl.when(i == 0)
    def _():
        st_ref[...] = jnp.zeros_like(st_ref)

    deg = jnp.sum(degp_ref[...], axis=0) + 1.0
    dinv = lax.rsqrt(deg)
    a = (acc0_ref[...] + acc1_ref[...] + hs_ref[...]) * dinv[:, None]
    a = a + b_ref[...]
    agg_ref[...] = a
    rid = lax.broadcasted_iota(jnp.int32, (BLK, 1), 0) + i * BLK
    a_m = jnp.where(rid < N, a, 0.0)
    st_ref[0:1, :] += jnp.sum(a_m, axis=0, keepdims=True)
    st_ref[1:2, :] += jnp.sum(a_m * a_m, axis=0, keepdims=True)


def _bn_body(agg_ref, x_ref, st_ref, g_ref, bt_ref, y_ref):
    inv_n = jnp.float32(1.0 / N)
    mean = st_ref[0:1, :] * inv_n
    ex2 = st_ref[1:2, :] * inv_n
    var = ex2 - mean * mean
    rstd = lax.rsqrt(var + 1e-5)
    bn = g_ref[...] * (agg_ref[...] - mean) * rstd + bt_ref[...]
    y_ref[...] = jnp.maximum(jnp.maximum(bn, 0.0) + x_ref[...], 0.0)


def kernel(x, edge_index, W, b, gamma, beta):
    src = edge_index[0].astype(jnp.int32)
    dst = edge_index[1].astype(jnp.int32)
    pad = E_PAD - E
    src1 = jnp.concatenate([src, jnp.zeros((pad,), jnp.int32)])
    dst1 = jnp.concatenate([dst, jnp.full((pad,), PAD_DST, jnp.int32)])
    dst2 = dst1.reshape(NCHT, CH)
    dst3 = dst1.reshape(NW, NCH, CH)
    x_p = jnp.concatenate([x, jnp.zeros((NP - N, D), jnp.float32)])

    degp = _deg_kernel(dst3)

    h = pl.pallas_call(
        _matmul_body,
        grid=(GRID,),
        in_specs=[
            pl.BlockSpec((BLK, D), lambda i: (i, 0)),
            pl.BlockSpec((D, D), lambda i: (0, 0)),
        ],
        out_specs=pl.BlockSpec((BLK, D), lambda i: (i, 0)),
        out_shape=jax.ShapeDtypeStruct((NP, D), jnp.float32),
    )(x_p, W)

    hs = pl.pallas_call(
        _scale_body,
        grid=(GRID,),
        in_specs=[
            pl.BlockSpec((BLK, D), lambda i: (i, 0)),
            pl.BlockSpec((NW, BLK), lambda i: (0, i)),
        ],
        out_specs=pl.BlockSpec((BLK, D), lambda i: (i, 0)),
        out_shape=jax.ShapeDtypeStruct((NP, D), jnp.float32),
    )(h, degp)

    accp = _scatter_kernel(hs, src1, dst2)

    agg, stats = pl.pallas_call(
        _agg_body,
        grid=(GRID,),
        in_specs=[
            pl.BlockSpec((BLK, D), lambda i: (i, 0)),
            pl.BlockSpec((BLK, D), lambda i: (i, 0)),
            pl.BlockSpec((BLK, D), lambda i: (i, 0)),
            pl.BlockSpec((NW, BLK), lambda i: (0, i)),
            pl.BlockSpec((1, D), lambda i: (0, 0)),
        ],
        out_specs=[
            pl.BlockSpec((BLK, D), lambda i: (i, 0)),
            pl.BlockSpec((2, D), lambda i: (0, 0)),
        ],
        out_shape=[
            jax.ShapeDtypeStruct((NP, D), jnp.float32),
            jax.ShapeDtypeStruct((2, D), jnp.float32),
        ],
    )(accp[0], accp[1], hs, degp, b.reshape(1, D))

    y = pl.pallas_call(
        _bn_body,
        grid=(GRID,),
        in_specs=[
            pl.BlockSpec((BLK, D), lambda i: (i, 0)),
            pl.BlockSpec((BLK, D), lambda i: (i, 0)),
            pl.BlockSpec((2, D), lambda i: (0, 0)),
            pl.BlockSpec((1, D), lambda i: (0, 0)),
            pl.BlockSpec((1, D), lambda i: (0, 0)),
        ],
        out_specs=pl.BlockSpec((BLK, D), lambda i: (i, 0)),
        out_shape=jax.ShapeDtypeStruct((NP, D), jnp.float32),
    )(agg, x_p, stats, gamma.reshape(1, D), beta.reshape(1, D))

    return y[:N]


# speedup vs baseline: 1.0635x; 1.0112x over previous
import functools

import jax
import jax.numpy as jnp
from jax import lax
from jax.experimental import pallas as pl
from jax.experimental.pallas import tpu as pltpu
from jax.experimental.pallas import tpu_sc as plsc

N = 10000
E = 320000
D = 128

NC = 2
NS = 16
NW = NC * NS
CH = 64
NCH = 160
GRP = 16
NG = NCH // GRP

EPT = NCH * CH
E_PAD = EPT * NW
PAD_DST = N

NCHT = E_PAD // CH
K0 = 256
K1 = NCHT // NS - K0
NG0 = K0 // GRP
NG1 = K1 // GRP
NCH0T = NS * K0

NA = 10240
RPT = NA // NS

NP = NA
BLK = 512
GRID = NP // BLK

_mesh = plsc.VectorSubcoreMesh(core_axis_name="c", subcore_axis_name="s")


@functools.partial(
    pl.kernel,
    out_type=jax.ShapeDtypeStruct((NW, NA), jnp.float32),
    mesh=_mesh,
    scratch_types=[
        pltpu.VMEM((GRP, CH), jnp.int32),
        pltpu.VMEM((NA,), jnp.float32),
    ],
    compiler_params=pltpu.CompilerParams(needs_layout_passes=False),
)
def _deg_kernel(dst_hbm, degp_hbm, dst_v, deg_v):
    c = lax.axis_index("c")
    s = lax.axis_index("s")
    wid = c * NS + s

    zeros16 = jnp.zeros((16,), jnp.float32)

    def zero_body(i, carry):
        deg_v[pl.ds(pl.multiple_of(i * 16, 16), 16)] = zeros16
        return carry

    lax.fori_loop(0, NA // 16, zero_body, 0)

    ones16 = jnp.ones((16,), jnp.float32)

    def grp_body(g, carry):
        goff = pl.multiple_of(g * GRP, GRP)
        pltpu.sync_copy(dst_hbm.at[wid, pl.ds(goff, GRP)], dst_v)

        def acc_body(j, carry2):
            for i in range(CH // 16):
                idx = dst_v[j, pl.ds(i * 16, 16)]
                plsc.addupdate_scatter(deg_v, [idx], ones16)
            return carry2

        lax.fori_loop(0, GRP, acc_body, 0)
        return carry

    lax.fori_loop(0, NG, grp_body, 0)

    pltpu.sync_copy(deg_v, degp_hbm.at[wid])


def _matmul_body(x_ref, w_ref, h_ref):
    h_ref[...] = jnp.dot(x_ref[...], w_ref[...],
                         preferred_element_type=jnp.float32,
                         precision=lax.Precision.HIGHEST)


def _scale_body(h_ref, degp_ref, hs_ref):
    deg = jnp.sum(degp_ref[...], axis=0) + 1.0
    dinv = lax.rsqrt(deg)
    hs_ref[...] = h_ref[...] * dinv[:, None]


@functools.partial(
    pl.kernel,
    out_type=jax.ShapeDtypeStruct((NC, NA, D), jnp.float32),
    mesh=_mesh,
    scratch_types=[
        pltpu.VMEM((GRP * CH,), jnp.int32),
        pltpu.VMEM((GRP * CH,), jnp.int32),
        pltpu.VMEM((GRP, CH), jnp.int32),
        pltpu.VMEM((GRP, CH), jnp.int32),
        pltpu.VMEM((CH, D), jnp.float32),
        pltpu.VMEM((CH, D), jnp.float32),
        pltpu.VMEM_SHARED((NA, D), jnp.float32),
        pltpu.SemaphoreType.DMA,
        pltpu.SemaphoreType.DMA,
        pltpu.SemaphoreType.DMA,
        pltpu.SemaphoreType.DMA,
        pltpu.SemaphoreType.DMA,
        pltpu.SemaphoreType.DMA,
    ],
    compiler_params=pltpu.CompilerParams(needs_layout_passes=False),
)
def _scatter_kernel(hs_hbm, src_hbm, dst_hbm, accp_hbm,
                    srcA_v, srcB_v, dstA_v, dstB_v, rows0_v, rows1_v,
                    acc_sh, sem0, sem1, semA, semB, semD0, semD1):
    c = lax.axis_index("c")
    s = lax.axis_index("s")

    ng = jnp.where(c == 0, NG0, NG1)
    base = jnp.where(c == 0, s * K0, NCH0T + s * K1)

    zeros16 = jnp.zeros((16,), jnp.float32)

    def zbody(r, carry):
        for i in range(D // 16):
            rows0_v[r, pl.ds(i * 16, 16)] = zeros16
        return carry

    lax.fori_loop(0, CH, zbody, 0)
    for k in range(RPT // CH):
        pltpu.sync_copy(rows0_v, acc_sh.at[pl.ds(s * RPT + k * CH, CH)])
    plsc.subcore_barrier()

    def src_slab(g):
        off = pl.multiple_of((base + g * GRP) * CH, GRP * CH)
        return src_hbm.at[pl.ds(off, GRP * CH)]

    def dst_slab(g):
        off = pl.multiple_of(base + g * GRP, GRP)
        return dst_hbm.at[pl.ds(off, GRP)]

    def sidx(buf, j):
        return buf.at[pl.ds(pl.multiple_of(j * CH, CH), CH)]

    pltpu.async_copy(src_slab(0), srcA_v, semA)
    pltpu.async_copy(dst_slab(0), dstA_v, semA)
    pltpu.make_async_copy(src_slab(0), srcA_v, semA).wait()
    pltpu.make_async_copy(dst_slab(0), dstA_v, semA).wait()
    pltpu.async_copy(hs_hbm.at[sidx(srcA_v, 0)], rows0_v, sem0)
    pltpu.async_copy(hs_hbm.at[sidx(srcA_v, 1)], rows1_v, sem1)

    def do_group(g, cs_v, cd_v, ns_v, nd_v, nsem):
        @pl.when(g + 1 < ng)
        def _():
            pltpu.async_copy(src_slab(g + 1), ns_v, nsem)
            pltpu.async_copy(dst_slab(g + 1), nd_v, nsem)

        def pair_body(p, carry):
            j0 = p * 2
            pltpu.make_async_copy(
                hs_hbm.at[sidx(cs_v, j0)], rows0_v, sem0).wait()
            pltpu.async_copy(rows0_v, acc_sh.at[cd_v.at[j0]], semD0, add=True)

            pltpu.make_async_copy(
                hs_hbm.at[sidx(cs_v, j0 + 1)], rows1_v, sem1).wait()
            pltpu.async_copy(rows1_v, acc_sh.at[cd_v.at[j0 + 1]], semD1,
                             add=True)

            pltpu.make_async_copy(
                rows0_v, acc_sh.at[cd_v.at[j0]], semD0).wait()

            @pl.when(p < GRP // 2 - 1)
            def _():
                pltpu.async_copy(hs_hbm.at[sidx(cs_v, j0 + 2)], rows0_v, sem0)

            @pl.when(jnp.logical_and(p == GRP // 2 - 1, g + 1 < ng))
            def _():
                pltpu.make_async_copy(src_slab(g + 1), ns_v, nsem).wait()
                pltpu.make_async_copy(dst_slab(g + 1), nd_v, nsem).wait()
                pltpu.async_copy(hs_hbm.at[sidx(ns_v, 0)], rows0_v, sem0)

            pltpu.make_async_copy(
                rows1_v, acc_sh.at[cd_v.at[j0 + 1]], semD1).wait()

            @pl.when(p < GRP // 2 - 1)
            def _():
                pltpu.async_copy(hs_hbm.at[sidx(cs_v, j0 + 3)], rows1_v, sem1)

            @pl.when(jnp.logical_and(p == GRP // 2 - 1, g + 1 < ng))
            def _():
                pltpu.async_copy(hs_hbm.at[sidx(ns_v, 1)], rows1_v, sem1)

            return carry

        lax.fori_loop(0, GRP // 2, pair_body, 0)

    def grp_body(g, carry):
        @pl.when(g % 2 == 0)
        def _():
            do_group(g, srcA_v, dstA_v, srcB_v, dstB_v, semB)

        @pl.when(g % 2 == 1)
        def _():
            do_group(g, srcB_v, dstB_v, srcA_v, dstA_v, semA)

        return carry

    lax.fori_loop(0, ng, grp_body, 0)

    plsc.subcore_barrier()
    pltpu.sync_copy(acc_sh.at[pl.ds(s * RPT, RPT)],
                    accp_hbm.at[c, pl.ds(s * RPT, RPT)])


def _agg_body(acc0_ref, acc1_ref, hs_ref, degp_ref, b_ref, agg_ref, st_ref):
    i = pl.program_id(0)

    @pl.when(i == 0)
    def _():
        st_ref[...] = jnp.zeros_like(st_ref)

    deg = jnp.sum(degp_ref[...], axis=0) + 1.0
    dinv = lax.rsqrt(deg)
    a = (acc0_ref[...] + acc1_ref[...] + hs_ref[...]) * dinv[:, None]
    a = a + b_ref[...]
    agg_ref[...] = a
    rid = lax.broadcasted_iota(jnp.int32, (BLK, 1), 0) + i * BLK
    a_m = jnp.where(rid < N, a, 0.0)
    st_ref[0:1, :] += jnp.sum(a_m, axis=0, keepdims=True)
    st_ref[1:2, :] += jnp.sum(a_m * a_m, axis=0, keepdims=True)


def _bn_body(agg_ref, x_ref, st_ref, g_ref, bt_ref, y_ref):
    inv_n = jnp.float32(1.0 / N)
    mean = st_ref[0:1, :] * inv_n
    ex2 = st_ref[1:2, :] * inv_n
    var = ex2 - mean * mean
    rstd = lax.rsqrt(var + 1e-5)
    bn = g_ref[...] * (agg_ref[...] - mean) * rstd + bt_ref[...]
    y_ref[...] = jnp.maximum(jnp.maximum(bn, 0.0) + x_ref[...], 0.0)


def kernel(x, edge_index, W, b, gamma, beta):
    src = edge_index[0].astype(jnp.int32)
    dst = edge_index[1].astype(jnp.int32)
    pad = E_PAD - E
    src1 = jnp.concatenate([src, jnp.zeros((pad,), jnp.int32)])
    dst1 = jnp.concatenate([dst, jnp.full((pad,), PAD_DST, jnp.int32)])
    dst2 = dst1.reshape(NCHT, CH)
    dst3 = dst1.reshape(NW, NCH, CH)
    x_p = jnp.concatenate([x, jnp.zeros((NP - N, D), jnp.float32)])

    degp = _deg_kernel(dst3)

    h = pl.pallas_call(
        _matmul_body,
        grid=(GRID,),
        in_specs=[
            pl.BlockSpec((BLK, D), lambda i: (i, 0)),
            pl.BlockSpec((D, D), lambda i: (0, 0)),
        ],
        out_specs=pl.BlockSpec((BLK, D), lambda i: (i, 0)),
        out_shape=jax.ShapeDtypeStruct((NP, D), jnp.float32),
    )(x_p, W)

    hs = pl.pallas_call(
        _scale_body,
        grid=(GRID,),
        in_specs=[
            pl.BlockSpec((BLK, D), lambda i: (i, 0)),
            pl.BlockSpec((NW, BLK), lambda i: (0, i)),
        ],
        out_specs=pl.BlockSpec((BLK, D), lambda i: (i, 0)),
        out_shape=jax.ShapeDtypeStruct((NP, D), jnp.float32),
    )(h, degp)

    accp = _scatter_kernel(hs, src1, dst2)

    agg, stats = pl.pallas_call(
        _agg_body,
        grid=(GRID,),
        in_specs=[
            pl.BlockSpec((BLK, D), lambda i: (i, 0)),
            pl.BlockSpec((BLK, D), lambda i: (i, 0)),
            pl.BlockSpec((BLK, D), lambda i: (i, 0)),
            pl.BlockSpec((NW, BLK), lambda i: (0, i)),
            pl.BlockSpec((1, D), lambda i: (0, 0)),
        ],
        out_specs=[
            pl.BlockSpec((BLK, D), lambda i: (i, 0)),
            pl.BlockSpec((2, D), lambda i: (0, 0)),
        ],
        out_shape=[
            jax.ShapeDtypeStruct((NP, D), jnp.float32),
            jax.ShapeDtypeStruct((2, D), jnp.float32),
        ],
    )(accp[0], accp[1], hs, degp, b.reshape(1, D))

    y = pl.pallas_call(
        _bn_body,
        grid=(GRID,),
        in_specs=[
            pl.BlockSpec((BLK, D), lambda i: (i, 0)),
            pl.BlockSpec((BLK, D), lambda i: (i, 0)),
            pl.BlockSpec((2, D), lambda i: (0, 0)),
            pl.BlockSpec((1, D), lambda i: (0, 0)),
            pl.BlockSpec((1, D), lambda i: (0, 0)),
        ],
        out_specs=pl.BlockSpec((BLK, D), lambda i: (i, 0)),
        out_shape=jax.ShapeDtypeStruct((NP, D), jnp.float32),
    )(agg, x_p, stats, gamma.reshape(1, D), beta.reshape(1, D))

    return y[:N]


# speedup vs baseline: 1.0833x; 1.0187x over previous
import functools

import jax
import jax.numpy as jnp
from jax import lax
from jax.experimental import pallas as pl
from jax.experimental.pallas import tpu as pltpu
from jax.experimental.pallas import tpu_sc as plsc

N = 10000
E = 320000
D = 128

NC = 2
NS = 16
NW = NC * NS
CH = 64
NCH = 160
GRP = 16
NG = NCH // GRP

EPT = NCH * CH
E_PAD = EPT * NW
PAD_DST = N

NCHT = E_PAD // CH
K0 = 272
K1 = NCHT // NS - K0
NG0 = K0 // GRP
NG1 = K1 // GRP
NCH0T = NS * K0

NA = 10240
RPT = NA // NS

NP = NA
BLK = 512
GRID = NP // BLK

_mesh = plsc.VectorSubcoreMesh(core_axis_name="c", subcore_axis_name="s")


@functools.partial(
    pl.kernel,
    out_type=jax.ShapeDtypeStruct((NW, NA), jnp.float32),
    mesh=_mesh,
    scratch_types=[
        pltpu.VMEM((GRP, CH), jnp.int32),
        pltpu.VMEM((NA,), jnp.float32),
    ],
    compiler_params=pltpu.CompilerParams(needs_layout_passes=False),
)
def _deg_kernel(dst_hbm, degp_hbm, dst_v, deg_v):
    c = lax.axis_index("c")
    s = lax.axis_index("s")
    wid = c * NS + s

    zeros16 = jnp.zeros((16,), jnp.float32)

    def zero_body(i, carry):
        deg_v[pl.ds(pl.multiple_of(i * 16, 16), 16)] = zeros16
        return carry

    lax.fori_loop(0, NA // 16, zero_body, 0)

    ones16 = jnp.ones((16,), jnp.float32)

    def grp_body(g, carry):
        goff = pl.multiple_of(g * GRP, GRP)
        pltpu.sync_copy(dst_hbm.at[wid, pl.ds(goff, GRP)], dst_v)

        def acc_body(j, carry2):
            for i in range(CH // 16):
                idx = dst_v[j, pl.ds(i * 16, 16)]
                plsc.addupdate_scatter(deg_v, [idx], ones16)
            return carry2

        lax.fori_loop(0, GRP, acc_body, 0)
        return carry

    lax.fori_loop(0, NG, grp_body, 0)

    pltpu.sync_copy(deg_v, degp_hbm.at[wid])


def _matmul_body(x_ref, w_ref, h_ref):
    h_ref[...] = jnp.dot(x_ref[...], w_ref[...],
                         preferred_element_type=jnp.float32,
                         precision=lax.Precision.HIGHEST)


def _scale_body(h_ref, degp_ref, hs_ref):
    deg = jnp.sum(degp_ref[...], axis=0) + 1.0
    dinv = lax.rsqrt(deg)
    hs_ref[...] = h_ref[...] * dinv[:, None]


@functools.partial(
    pl.kernel,
    out_type=jax.ShapeDtypeStruct((NC, NA, D), jnp.float32),
    mesh=_mesh,
    scratch_types=[
        pltpu.VMEM((GRP * CH,), jnp.int32),
        pltpu.VMEM((GRP * CH,), jnp.int32),
        pltpu.VMEM((GRP, CH), jnp.int32),
        pltpu.VMEM((GRP, CH), jnp.int32),
        pltpu.VMEM((CH, D), jnp.float32),
        pltpu.VMEM((CH, D), jnp.float32),
        pltpu.VMEM_SHARED((NA, D), jnp.float32),
        pltpu.SemaphoreType.DMA,
        pltpu.SemaphoreType.DMA,
        pltpu.SemaphoreType.DMA,
        pltpu.SemaphoreType.DMA,
        pltpu.SemaphoreType.DMA,
        pltpu.SemaphoreType.DMA,
    ],
    compiler_params=pltpu.CompilerParams(needs_layout_passes=False),
)
def _scatter_kernel(hs_hbm, src_hbm, dst_hbm, accp_hbm,
                    srcA_v, srcB_v, dstA_v, dstB_v, rows0_v, rows1_v,
                    acc_sh, sem0, sem1, semA, semB, semD0, semD1):
    c = lax.axis_index("c")
    s = lax.axis_index("s")

    ng = jnp.where(c == 0, NG0, NG1)
    base = jnp.where(c == 0, s * K0, NCH0T + s * K1)

    zeros16 = jnp.zeros((16,), jnp.float32)

    def zbody(r, carry):
        for i in range(D // 16):
            rows0_v[r, pl.ds(i * 16, 16)] = zeros16
        return carry

    lax.fori_loop(0, CH, zbody, 0)
    for k in range(RPT // CH):
        pltpu.sync_copy(rows0_v, acc_sh.at[pl.ds(s * RPT + k * CH, CH)])
    plsc.subcore_barrier()

    def src_slab(g):
        off = pl.multiple_of((base + g * GRP) * CH, GRP * CH)
        return src_hbm.at[pl.ds(off, GRP * CH)]

    def dst_slab(g):
        off = pl.multiple_of(base + g * GRP, GRP)
        return dst_hbm.at[pl.ds(off, GRP)]

    def sidx(buf, j):
        return buf.at[pl.ds(pl.multiple_of(j * CH, CH), CH)]

    pltpu.async_copy(src_slab(0), srcA_v, semA)
    pltpu.async_copy(dst_slab(0), dstA_v, semA)
    pltpu.make_async_copy(src_slab(0), srcA_v, semA).wait()
    pltpu.make_async_copy(dst_slab(0), dstA_v, semA).wait()
    pltpu.async_copy(hs_hbm.at[sidx(srcA_v, 0)], rows0_v, sem0)
    pltpu.async_copy(hs_hbm.at[sidx(srcA_v, 1)], rows1_v, sem1)

    def do_group(g, cs_v, cd_v, ns_v, nd_v, nsem):
        @pl.when(g + 1 < ng)
        def _():
            pltpu.async_copy(src_slab(g + 1), ns_v, nsem)
            pltpu.async_copy(dst_slab(g + 1), nd_v, nsem)

        def pair_body(p, carry):
            j0 = p * 2
            pltpu.make_async_copy(
                hs_hbm.at[sidx(cs_v, j0)], rows0_v, sem0).wait()
            pltpu.async_copy(rows0_v, acc_sh.at[cd_v.at[j0]], semD0, add=True)

            pltpu.make_async_copy(
                hs_hbm.at[sidx(cs_v, j0 + 1)], rows1_v, sem1).wait()
            pltpu.async_copy(rows1_v, acc_sh.at[cd_v.at[j0 + 1]], semD1,
                             add=True)

            pltpu.make_async_copy(
                rows0_v, acc_sh.at[cd_v.at[j0]], semD0).wait()

            @pl.when(p < GRP // 2 - 1)
            def _():
                pltpu.async_copy(hs_hbm.at[sidx(cs_v, j0 + 2)], rows0_v, sem0)

            @pl.when(jnp.logical_and(p == GRP // 2 - 1, g + 1 < ng))
            def _():
                pltpu.make_async_copy(src_slab(g + 1), ns_v, nsem).wait()
                pltpu.make_async_copy(dst_slab(g + 1), nd_v, nsem).wait()
                pltpu.async_copy(hs_hbm.at[sidx(ns_v, 0)], rows0_v, sem0)

            pltpu.make_async_copy(
                rows1_v, acc_sh.at[cd_v.at[j0 + 1]], semD1).wait()

            @pl.when(p < GRP // 2 - 1)
            def _():
                pltpu.async_copy(hs_hbm.at[sidx(cs_v, j0 + 3)], rows1_v, sem1)

            @pl.when(jnp.logical_and(p == GRP // 2 - 1, g + 1 < ng))
            def _():
                pltpu.async_copy(hs_hbm.at[sidx(ns_v, 1)], rows1_v, sem1)

            return carry

        lax.fori_loop(0, GRP // 2, pair_body, 0)

    def grp_body(g, carry):
        @pl.when(g % 2 == 0)
        def _():
            do_group(g, srcA_v, dstA_v, srcB_v, dstB_v, semB)

        @pl.when(g % 2 == 1)
        def _():
            do_group(g, srcB_v, dstB_v, srcA_v, dstA_v, semA)

        return carry

    lax.fori_loop(0, ng, grp_body, 0)

    plsc.subcore_barrier()
    pltpu.sync_copy(acc_sh.at[pl.ds(s * RPT, RPT)],
                    accp_hbm.at[c, pl.ds(s * RPT, RPT)])


def _agg_body(acc0_ref, acc1_ref, hs_ref, degp_ref, b_ref, agg_ref, st_ref):
    i = pl.program_id(0)

    @pl.when(i == 0)
    def _():
        st_ref[...] = jnp.zeros_like(st_ref)

    deg = jnp.sum(degp_ref[...], axis=0) + 1.0
    dinv = lax.rsqrt(deg)
    a = (acc0_ref[...] + acc1_ref[...] + hs_ref[...]) * dinv[:, None]
    a = a + b_ref[...]
    agg_ref[...] = a
    rid = lax.broadcasted_iota(jnp.int32, (BLK, 1), 0) + i * BLK
    a_m = jnp.where(rid < N, a, 0.0)
    st_ref[0:1, :] += jnp.sum(a_m, axis=0, keepdims=True)
    st_ref[1:2, :] += jnp.sum(a_m * a_m, axis=0, keepdims=True)


def _bn_body(agg_ref, x_ref, st_ref, g_ref, bt_ref, y_ref):
    inv_n = jnp.float32(1.0 / N)
    mean = st_ref[0:1, :] * inv_n
    ex2 = st_ref[1:2, :] * inv_n
    var = ex2 - mean * mean
    rstd = lax.rsqrt(var + 1e-5)
    bn = g_ref[...] * (agg_ref[...] - mean) * rstd + bt_ref[...]
    y_ref[...] = jnp.maximum(jnp.maximum(bn, 0.0) + x_ref[...], 0.0)


def kernel(x, edge_index, W, b, gamma, beta):
    src = edge_index[0].astype(jnp.int32)
    dst = edge_index[1].astype(jnp.int32)
    pad = E_PAD - E
    src1 = jnp.concatenate([src, jnp.zeros((pad,), jnp.int32)])
    dst1 = jnp.concatenate([dst, jnp.full((pad,), PAD_DST, jnp.int32)])
    dst2 = dst1.reshape(NCHT, CH)
    dst3 = dst1.reshape(NW, NCH, CH)
    x_p = jnp.concatenate([x, jnp.zeros((NP - N, D), jnp.float32)])

    degp = _deg_kernel(dst3)

    h = pl.pallas_call(
        _matmul_body,
        grid=(GRID,),
        in_specs=[
            pl.BlockSpec((BLK, D), lambda i: (i, 0)),
            pl.BlockSpec((D, D), lambda i: (0, 0)),
        ],
        out_specs=pl.BlockSpec((BLK, D), lambda i: (i, 0)),
        out_shape=jax.ShapeDtypeStruct((NP, D), jnp.float32),
    )(x_p, W)

    hs = pl.pallas_call(
        _scale_body,
        grid=(GRID,),
        in_specs=[
            pl.BlockSpec((BLK, D), lambda i: (i, 0)),
            pl.BlockSpec((NW, BLK), lambda i: (0, i)),
        ],
        out_specs=pl.BlockSpec((BLK, D), lambda i: (i, 0)),
        out_shape=jax.ShapeDtypeStruct((NP, D), jnp.float32),
    )(h, degp)

    accp = _scatter_kernel(hs, src1, dst2)

    agg, stats = pl.pallas_call(
        _agg_body,
        grid=(GRID,),
        in_specs=[
            pl.BlockSpec((BLK, D), lambda i: (i, 0)),
            pl.BlockSpec((BLK, D), lambda i: (i, 0)),
            pl.BlockSpec((BLK, D), lambda i: (i, 0)),
            pl.BlockSpec((NW, BLK), lambda i: (0, i)),
            pl.BlockSpec((1, D), lambda i: (0, 0)),
        ],
        out_specs=[
            pl.BlockSpec((BLK, D), lambda i: (i, 0)),
            pl.BlockSpec((2, D), lambda i: (0, 0)),
        ],
        out_shape=[
            jax.ShapeDtypeStruct((NP, D), jnp.float32),
            jax.ShapeDtypeStruct((2, D), jnp.float32),
        ],
    )(accp[0], accp[1], hs, degp, b.reshape(1, D))

    y = pl.pallas_call(
        _bn_body,
        grid=(GRID,),
        in_specs=[
            pl.BlockSpec((BLK, D), lambda i: (i, 0)),
            pl.BlockSpec((BLK, D), lambda i: (i, 0)),
            pl.BlockSpec((2, D), lambda i: (0, 0)),
            pl.BlockSpec((1, D), lambda i: (0, 0)),
            pl.BlockSpec((1, D), lambda i: (0, 0)),
        ],
        out_specs=pl.BlockSpec((BLK, D), lambda i: (i, 0)),
        out_shape=jax.ShapeDtypeStruct((NP, D), jnp.float32),
    )(agg, x_p, stats, gamma.reshape(1, D), beta.reshape(1, D))

    return y[:N]


# speedup vs baseline: 1.1204x; 1.0342x over previous
import functools

import jax
import jax.numpy as jnp
from jax import lax
from jax.experimental import pallas as pl
from jax.experimental.pallas import tpu as pltpu
from jax.experimental.pallas import tpu_sc as plsc

N = 10000
E = 320000
D = 128

NC = 2
NS = 16
NW = NC * NS
CH = 64
NCH = 160
GRP = 16
NG = NCH // GRP

EPT = NCH * CH
E_PAD = EPT * NW
PAD_DST = N

NCHT = E_PAD // CH
K0 = 288
K1 = NCHT // NS - K0
NG0 = K0 // GRP
NG1 = K1 // GRP
NCH0T = NS * K0

NA = 10240
RPT = NA // NS

NP = NA
BLK = 512
GRID = NP // BLK

_mesh = plsc.VectorSubcoreMesh(core_axis_name="c", subcore_axis_name="s")


@functools.partial(
    pl.kernel,
    out_type=jax.ShapeDtypeStruct((NW, NA), jnp.float32),
    mesh=_mesh,
    scratch_types=[
        pltpu.VMEM((GRP, CH), jnp.int32),
        pltpu.VMEM((NA,), jnp.float32),
    ],
    compiler_params=pltpu.CompilerParams(needs_layout_passes=False),
)
def _deg_kernel(dst_hbm, degp_hbm, dst_v, deg_v):
    c = lax.axis_index("c")
    s = lax.axis_index("s")
    wid = c * NS + s

    zeros16 = jnp.zeros((16,), jnp.float32)

    def zero_body(i, carry):
        deg_v[pl.ds(pl.multiple_of(i * 16, 16), 16)] = zeros16
        return carry

    lax.fori_loop(0, NA // 16, zero_body, 0)

    ones16 = jnp.ones((16,), jnp.float32)

    def grp_body(g, carry):
        goff = pl.multiple_of(g * GRP, GRP)
        pltpu.sync_copy(dst_hbm.at[wid, pl.ds(goff, GRP)], dst_v)

        def acc_body(j, carry2):
            for i in range(CH // 16):
                idx = dst_v[j, pl.ds(i * 16, 16)]
                plsc.addupdate_scatter(deg_v, [idx], ones16)
            return carry2

        lax.fori_loop(0, GRP, acc_body, 0)
        return carry

    lax.fori_loop(0, NG, grp_body, 0)

    pltpu.sync_copy(deg_v, degp_hbm.at[wid])


def _matmul_body(x_ref, w_ref, h_ref):
    h_ref[...] = jnp.dot(x_ref[...], w_ref[...],
                         preferred_element_type=jnp.float32,
                         precision=lax.Precision.HIGHEST)


def _scale_body(h_ref, degp_ref, hs_ref):
    deg = jnp.sum(degp_ref[...], axis=0) + 1.0
    dinv = lax.rsqrt(deg)
    hs_ref[...] = h_ref[...] * dinv[:, None]


@functools.partial(
    pl.kernel,
    out_type=jax.ShapeDtypeStruct((NC, NA, D), jnp.float32),
    mesh=_mesh,
    scratch_types=[
        pltpu.VMEM((GRP * CH,), jnp.int32),
        pltpu.VMEM((GRP * CH,), jnp.int32),
        pltpu.VMEM((GRP, CH), jnp.int32),
        pltpu.VMEM((GRP, CH), jnp.int32),
        pltpu.VMEM((CH, D), jnp.float32),
        pltpu.VMEM((CH, D), jnp.float32),
        pltpu.VMEM_SHARED((NA, D), jnp.float32),
        pltpu.SemaphoreType.DMA,
        pltpu.SemaphoreType.DMA,
        pltpu.SemaphoreType.DMA,
        pltpu.SemaphoreType.DMA,
        pltpu.SemaphoreType.DMA,
        pltpu.SemaphoreType.DMA,
    ],
    compiler_params=pltpu.CompilerParams(needs_layout_passes=False),
)
def _scatter_kernel(hs_hbm, src_hbm, dst_hbm, accp_hbm,
                    srcA_v, srcB_v, dstA_v, dstB_v, rows0_v, rows1_v,
                    acc_sh, sem0, sem1, semA, semB, semD0, semD1):
    c = lax.axis_index("c")
    s = lax.axis_index("s")

    ng = jnp.where(c == 0, NG0, NG1)
    base = jnp.where(c == 0, s * K0, NCH0T + s * K1)

    zeros16 = jnp.zeros((16,), jnp.float32)

    def zbody(r, carry):
        for i in range(D // 16):
            rows0_v[r, pl.ds(i * 16, 16)] = zeros16
        return carry

    lax.fori_loop(0, CH, zbody, 0)
    for k in range(RPT // CH):
        pltpu.sync_copy(rows0_v, acc_sh.at[pl.ds(s * RPT + k * CH, CH)])
    plsc.subcore_barrier()

    def src_slab(g):
        off = pl.multiple_of((base + g * GRP) * CH, GRP * CH)
        return src_hbm.at[pl.ds(off, GRP * CH)]

    def dst_slab(g):
        off = pl.multiple_of(base + g * GRP, GRP)
        return dst_hbm.at[pl.ds(off, GRP)]

    def sidx(buf, j):
        return buf.at[pl.ds(pl.multiple_of(j * CH, CH), CH)]

    pltpu.async_copy(src_slab(0), srcA_v, semA)
    pltpu.async_copy(dst_slab(0), dstA_v, semA)
    pltpu.make_async_copy(src_slab(0), srcA_v, semA).wait()
    pltpu.make_async_copy(dst_slab(0), dstA_v, semA).wait()
    pltpu.async_copy(hs_hbm.at[sidx(srcA_v, 0)], rows0_v, sem0)
    pltpu.async_copy(hs_hbm.at[sidx(srcA_v, 1)], rows1_v, sem1)

    def do_group(g, cs_v, cd_v, ns_v, nd_v, nsem):
        @pl.when(g + 1 < ng)
        def _():
            pltpu.async_copy(src_slab(g + 1), ns_v, nsem)
            pltpu.async_copy(dst_slab(g + 1), nd_v, nsem)

        def pair_body(p, carry):
            j0 = p * 2
            pltpu.make_async_copy(
                hs_hbm.at[sidx(cs_v, j0)], rows0_v, sem0).wait()
            pltpu.async_copy(rows0_v, acc_sh.at[cd_v.at[j0]], semD0, add=True)

            pltpu.make_async_copy(
                hs_hbm.at[sidx(cs_v, j0 + 1)], rows1_v, sem1).wait()
            pltpu.async_copy(rows1_v, acc_sh.at[cd_v.at[j0 + 1]], semD1,
                             add=True)

            pltpu.make_async_copy(
                rows0_v, acc_sh.at[cd_v.at[j0]], semD0).wait()

            @pl.when(p < GRP // 2 - 1)
            def _():
                pltpu.async_copy(hs_hbm.at[sidx(cs_v, j0 + 2)], rows0_v, sem0)

            @pl.when(jnp.logical_and(p == GRP // 2 - 1, g + 1 < ng))
            def _():
                pltpu.make_async_copy(src_slab(g + 1), ns_v, nsem).wait()
                pltpu.make_async_copy(dst_slab(g + 1), nd_v, nsem).wait()
                pltpu.async_copy(hs_hbm.at[sidx(ns_v, 0)], rows0_v, sem0)

            pltpu.make_async_copy(
                rows1_v, acc_sh.at[cd_v.at[j0 + 1]], semD1).wait()

            @pl.when(p < GRP // 2 - 1)
            def _():
                pltpu.async_copy(hs_hbm.at[sidx(cs_v, j0 + 3)], rows1_v, sem1)

            @pl.when(jnp.logical_and(p == GRP // 2 - 1, g + 1 < ng))
            def _():
                pltpu.async_copy(hs_hbm.at[sidx(ns_v, 1)], rows1_v, sem1)

            return carry

        lax.fori_loop(0, GRP // 2, pair_body, 0)

    def grp_body(g, carry):
        @pl.when(g % 2 == 0)
        def _():
            do_group(g, srcA_v, dstA_v, srcB_v, dstB_v, semB)

        @pl.when(g % 2 == 1)
        def _():
            do_group(g, srcB_v, dstB_v, srcA_v, dstA_v, semA)

        return carry

    lax.fori_loop(0, ng, grp_body, 0)

    plsc.subcore_barrier()
    pltpu.sync_copy(acc_sh.at[pl.ds(s * RPT, RPT)],
                    accp_hbm.at[c, pl.ds(s * RPT, RPT)])


def _agg_body(acc0_ref, acc1_ref, hs_ref, degp_ref, b_ref, agg_ref, st_ref):
    i = pl.program_id(0)

    @pl.when(i == 0)
    def _():
        st_ref[...] = jnp.zeros_like(st_ref)

    deg = jnp.sum(degp_ref[...], axis=0) + 1.0
    dinv = lax.rsqrt(deg)
    a = (acc0_ref[...] + acc1_ref[...] + hs_ref[...]) * dinv[:, None]
    a = a + b_ref[...]
    agg_ref[...] = a
    rid = lax.broadcasted_iota(jnp.int32, (BLK, 1), 0) + i * BLK
    a_m = jnp.where(rid < N, a, 0.0)
    st_ref[0:1, :] += jnp.sum(a_m, axis=0, keepdims=True)
    st_ref[1:2, :] += jnp.sum(a_m * a_m, axis=0, keepdims=True)


def _bn_body(agg_ref, x_ref, st_ref, g_ref, bt_ref, y_ref):
    inv_n = jnp.float32(1.0 / N)
    mean = st_ref[0:1, :] * inv_n
    ex2 = st_ref[1:2, :] * inv_n
    var = ex2 - mean * mean
    rstd = lax.rsqrt(var + 1e-5)
    bn = g_ref[...] * (agg_ref[...] - mean) * rstd + bt_ref[...]
    y_ref[...] = jnp.maximum(jnp.maximum(bn, 0.0) + x_ref[...], 0.0)


def kernel(x, edge_index, W, b, gamma, beta):
    src = edge_index[0].astype(jnp.int32)
    dst = edge_index[1].astype(jnp.int32)
    pad = E_PAD - E
    src1 = jnp.concatenate([src, jnp.zeros((pad,), jnp.int32)])
    dst1 = jnp.concatenate([dst, jnp.full((pad,), PAD_DST, jnp.int32)])
    dst2 = dst1.reshape(NCHT, CH)
    dst3 = dst1.reshape(NW, NCH, CH)
    x_p = jnp.concatenate([x, jnp.zeros((NP - N, D), jnp.float32)])

    degp = _deg_kernel(dst3)

    h = pl.pallas_call(
        _matmul_body,
        grid=(GRID,),
        in_specs=[
            pl.BlockSpec((BLK, D), lambda i: (i, 0)),
            pl.BlockSpec((D, D), lambda i: (0, 0)),
        ],
        out_specs=pl.BlockSpec((BLK, D), lambda i: (i, 0)),
        out_shape=jax.ShapeDtypeStruct((NP, D), jnp.float32),
    )(x_p, W)

    hs = pl.pallas_call(
        _scale_body,
        grid=(GRID,),
        in_specs=[
            pl.BlockSpec((BLK, D), lambda i: (i, 0)),
            pl.BlockSpec((NW, BLK), lambda i: (0, i)),
        ],
        out_specs=pl.BlockSpec((BLK, D), lambda i: (i, 0)),
        out_shape=jax.ShapeDtypeStruct((NP, D), jnp.float32),
    )(h, degp)

    accp = _scatter_kernel(hs, src1, dst2)

    agg, stats = pl.pallas_call(
        _agg_body,
        grid=(GRID,),
        in_specs=[
            pl.BlockSpec((BLK, D), lambda i: (i, 0)),
            pl.BlockSpec((BLK, D), lambda i: (i, 0)),
            pl.BlockSpec((BLK, D), lambda i: (i, 0)),
            pl.BlockSpec((NW, BLK), lambda i: (0, i)),
            pl.BlockSpec((1, D), lambda i: (0, 0)),
        ],
        out_specs=[
            pl.BlockSpec((BLK, D), lambda i: (i, 0)),
            pl.BlockSpec((2, D), lambda i: (0, 0)),
        ],
        out_shape=[
            jax.ShapeDtypeStruct((NP, D), jnp.float32),
            jax.ShapeDtypeStruct((2, D), jnp.float32),
        ],
    )(accp[0], accp[1], hs, degp, b.reshape(1, D))

    y = pl.pallas_call(
        _bn_body,
        grid=(GRID,),
        in_specs=[
            pl.BlockSpec((BLK, D), lambda i: (i, 0)),
            pl.BlockSpec((BLK, D), lambda i: (i, 0)),
            pl.BlockSpec((2, D), lambda i: (0, 0)),
            pl.BlockSpec((1, D), lambda i: (0, 0)),
            pl.BlockSpec((1, D), lambda i: (0, 0)),
        ],
        out_specs=pl.BlockSpec((BLK, D), lambda i: (i, 0)),
        out_shape=jax.ShapeDtypeStruct((NP, D), jnp.float32),
    )(agg, x_p, stats, gamma.reshape(1, D), beta.reshape(1, D))

    return y[:N]
